# baseline (speedup 1.0000x reference)
import math
from itertools import combinations

import jax
import jax.numpy as jnp
import numpy as np
from jax import lax
from jax.experimental import pallas as pl
from jax.experimental.pallas import tpu as pltpu
from jax.experimental.pallas import tpu_sc as plsc

D = 32
K = 6
G = 100
_NS = np.arange(G)
N = int(_NS.sum())
P_TOTAL = int(sum(int(n) * (int(n) - 1) // 2 for n in _NS))

NPAD = 4992
BR = 104
NW = 32
CHUNK = 128
NCH = 40
EPAD = NW * NCH * CHUNK
BE = 2048

_pairs = []
_gids = []
for _gi, _n in enumerate(_NS):
    _n = int(_n)
    if _n >= 2:
        _p = np.array(list(combinations(range(_n), 2)), dtype=np.int64)
        _pairs.append(_p)
        _gids.append(np.full(_p.shape[0], _gi, dtype=np.int64))
_pairs = np.concatenate(_pairs, 0)
_gids = np.concatenate(_gids, 0)
_offsets = np.cumsum(_NS) - _NS
_SRC = (_pairs[:, 0] + _offsets[_gids]).astype(np.int32)
_DST = (_pairs[:, 1] + _offsets[_gids]).astype(np.int32)
_SRC_PAD = np.zeros(EPAD, np.int32)
_DST_PAD = np.zeros(EPAD, np.int32)
_SRC_PAD[:P_TOTAL] = _SRC
_DST_PAD[:P_TOTAL] = _DST


def _lrelu(x):
    return jnp.where(x >= 0, x, 0.1 * x)


def _ln(x, g, b):
    mu = jnp.mean(x, axis=-1, keepdims=True)
    var = jnp.mean((x - mu) ** 2, axis=-1, keepdims=True)
    return (x - mu) / jnp.sqrt(var + 1e-5) * g + b


def _precast_body(yr_ref, yi_ref, br_ref, bi_ref, bs_ref):
    yr = yr_ref[...]
    yi = yi_ref[...]
    br_ref[...] = yr.astype(jnp.bfloat16)
    bi_ref[...] = yi.astype(jnp.bfloat16)
    bs_ref[...] = (yr + yi).astype(jnp.bfloat16)


def _precast(yr, yi):
    bs = jax.ShapeDtypeStruct((N, N), jnp.bfloat16)
    return pl.pallas_call(
        _precast_body,
        grid=(NPAD // BR,),
        in_specs=[pl.BlockSpec((BR, N), lambda i: (i, 0)),
                  pl.BlockSpec((BR, N), lambda i: (i, 0))],
        out_specs=[pl.BlockSpec((BR, N), lambda i: (i, 0))] * 3,
        out_shape=(bs, bs, bs),
    )(yr, yi)


def _mv_body(vth_ref, br_ref, bi_ref, bs_ref, out_ref):
    v = vth_ref[:, 0:1]
    th = vth_ref[:, 1:2]
    vr = (v * jnp.cos(th))[:N, :]
    vi = (v * jnp.sin(th))[:N, :]
    kw = dict(preferred_element_type=jnp.float32)
    t1 = jnp.dot(bs_ref[...], vr, **kw)
    t2 = jnp.dot(br_ref[...], vi - vr, **kw)
    t3 = jnp.dot(bi_ref[...], vr + vi, **kw)
    out_ref[...] = jnp.concatenate([t1 - t3, t1 + t2], 1)


def _matvec(vth, br, bi, bs):
    return pl.pallas_call(
        _mv_body,
        grid=(NPAD // BR,),
        in_specs=[
            pl.BlockSpec((NPAD, 2), lambda i: (0, 0)),
            pl.BlockSpec((BR, N), lambda i: (i, 0)),
            pl.BlockSpec((BR, N), lambda i: (i, 0)),
            pl.BlockSpec((BR, N), lambda i: (i, 0)),
        ],
        out_specs=pl.BlockSpec((BR, 2), lambda i: (i, 0)),
        out_shape=jax.ShapeDtypeStruct((NPAD, 2), jnp.float32),
    )(vth, br, bi, bs)


def _edge_body(mdst_ref, ef_ref, W1, b1, g1, be1, W2, b2, g2, be2, W3, b3,
               out_ref):
    x = jnp.concatenate([mdst_ref[...], ef_ref[:, 0:3]], 1)
    h = _lrelu(_ln(jnp.dot(x, W1[...], preferred_element_type=jnp.float32)
                   + b1[...], g1[...], be1[...]))
    h = _lrelu(_ln(jnp.dot(h, W2[...], preferred_element_type=jnp.float32)
                   + b2[...], g2[...], be2[...]))
    phi = jnp.dot(h, W3[...], preferred_element_type=jnp.float32) + b3[...]
    out_ref[...] = phi * ef_ref[:, 3:4]


def _edge_mlp(mdst, ef4, p):
    full = lambda s: pl.BlockSpec(s, lambda i: (0,) * len(s))
    return pl.pallas_call(
        _edge_body,
        grid=(EPAD // BE,),
        in_specs=[
            pl.BlockSpec((BE, D), lambda i: (i, 0)),
            pl.BlockSpec((BE, 4), lambda i: (i, 0)),
            full((D + 3, D + 3)), full((1, D + 3)), full((1, D + 3)),
            full((1, D + 3)),
            full((D + 3, D + 3)), full((1, D + 3)), full((1, D + 3)),
            full((1, D + 3)),
            full((D + 3, D)), full((1, D)),
        ],
        out_specs=pl.BlockSpec((BE, D), lambda i: (i, 0)),
        out_shape=jax.ShapeDtypeStruct((EPAD, D), jnp.float32),
    )(mdst, ef4,
      p['W1'], p['b1'].reshape(1, -1), p['g1'].reshape(1, -1),
      p['be1'].reshape(1, -1),
      p['W2'], p['b2'].reshape(1, -1), p['g2'].reshape(1, -1),
      p['be2'].reshape(1, -1),
      p['W3'], p['b3'].reshape(1, -1))


def _prepA_body(degp_ref, a_ref):
    deg = degp_ref[0, :, 0:1] + degp_ref[1, :, 0:1]
    a_ref[...] = 1.0 / jnp.maximum(deg, 1.0)


def _prep_A(deg_parts):
    return pl.pallas_call(
        _prepA_body,
        out_shape=jax.ShapeDtypeStruct((NPAD, 1), jnp.float32),
    )(deg_parts)


def _node_body(vth_ref, m_ref, iout_ref, c_ref, a_ref, aggp_ref,
               tW1, tb1, tg1, tbe1, tW2, tb2, tg2, tbe2, tW3, tb3,
               vW1, vb1, vg1, vbe1, vW2, vb2, vg2, vbe2, vW3, vb3,
               mW1, mb1, mg1, mbe1, mW2, mb2, mg2, mbe2, mW3, mb3,
               vth_out, m_out, loss_out):
    v = vth_ref[:, 0:1]
    th = vth_ref[:, 1:2]
    ir = iout_ref[:, 0:1]
    ii = iout_ref[:, 1:2]
    P = c_ref[:, 0:1]
    Q = c_ref[:, 1:2]
    slack = c_ref[:, 2:3]
    pv = c_ref[:, 3:4]
    valid = c_ref[:, 4:5]
    vr = v * jnp.cos(th)
    vi = v * jnp.sin(th)
    sr = vr * ir + vi * ii
    si = vi * ir - vr * ii
    ok = valid > 0.0
    dP = jnp.where(ok, (1.0 - slack) * (P - sr), 0.0)
    dQ = jnp.where(ok, (1.0 - slack) * (1.0 - pv) * (Q - si), 0.0)
    agg = (aggp_ref[0] + aggp_ref[1]) * a_ref[...]
    feats = jnp.concatenate([v, th, dP, dQ, m_ref[...], agg], 1)

    def blk(W1, b1, g1, be1, W2, b2, g2, be2, W3, b3):
        h = _lrelu(_ln(jnp.dot(feats, W1[...],
                               preferred_element_type=jnp.float32)
                       + b1[...], g1[...], be1[...]))
        h = _lrelu(_ln(jnp.dot(h, W2[...],
                               preferred_element_type=jnp.float32)
                       + b2[...], g2[...], be2[...]))
        return jnp.dot(h, W3[...], preferred_element_type=jnp.float32) + b3[...]

    dth = (1.0 - slack) * blk(tW1, tb1, tg1, tbe1, tW2, tb2, tg2, tbe2,
                              tW3, tb3)
    dv = (1.0 - slack) * (1.0 - pv) * blk(vW1, vb1, vg1, vbe1, vW2, vb2,
                                          vg2, vbe2, vW3, vb3)
    dm = _ln(jnp.tanh(blk(mW1, mb1, mg1, mbe1, mW2, mb2, mg2, mbe2,
                          mW3, mb3)), 1.0, 0.0)
    th_new = jnp.mod(th + dth + math.pi, 2.0 * math.pi) - math.pi
    v_new = jnp.clip(v + dv, 0.4, 1.2)
    vth_out[...] = jnp.concatenate([v_new, th_new], 1)
    m_out[...] = m_ref[...] + dm
    loss_out[...] = jnp.sum((dP * dP + dQ * dQ) * valid,
                            keepdims=True).reshape(1, 1) / float(N)


def _node_stage(vth, m, iout, consts, A, agg_parts, pt, pv_, pm):
    def flat(p):
        return (p['W1'], p['b1'].reshape(1, -1), p['g1'].reshape(1, -1),
                p['be1'].reshape(1, -1),
                p['W2'], p['b2'].reshape(1, -1), p['g2'].reshape(1, -1),
                p['be2'].reshape(1, -1),
                p['W3'], p['b3'].reshape(1, -1))
    return pl.pallas_call(
        _node_body,
        out_shape=(
            jax.ShapeDtypeStruct((NPAD, 2), jnp.float32),
            jax.ShapeDtypeStruct((NPAD, D), jnp.float32),
            jax.ShapeDtypeStruct((1, 1), jnp.float32),
        ),
    )(vth, m, iout, consts, A, agg_parts, *flat(pt), *flat(pv_), *flat(pm))


_EW = NCH * CHUNK
_NROW = NPAD // 16


def _sc_mesh():
    return plsc.VectorSubcoreMesh(core_axis_name="c", subcore_axis_name="s")


def _gather_body(m_hbm, idx_hbm, out_hbm, idx_v, rows_v, sem):
    c = lax.axis_index("c")
    s = lax.axis_index("s")
    wid = s * 2 + c
    pltpu.sync_copy(idx_hbm.at[wid], idx_v)
    base = wid * _EW

    def chunk(j, carry):
        pltpu.async_copy(m_hbm.at[idx_v.at[j]], rows_v, sem).wait()
        pltpu.sync_copy(rows_v, out_hbm.at[pl.ds(base + j * CHUNK, CHUNK)])
        return carry

    lax.fori_loop(0, NCH, chunk, 0, unroll=False)


def _sc_gather(m, dst3):
    import functools
    k = functools.partial(
        pl.kernel,
        out_type=jax.ShapeDtypeStruct((EPAD, D), jnp.float32),
        mesh=_sc_mesh(),
        compiler_params=pltpu.CompilerParams(use_tc_tiling_on_sc=False),
        scratch_types=[
            pltpu.VMEM((NCH, CHUNK), jnp.int32),
            pltpu.VMEM((CHUNK, D), jnp.float32),
            pltpu.SemaphoreType.DMA,
        ],
    )(_gather_body)
    return k(m, dst3)


def _scatter_body(phi_hbm, src_hbm, dst_hbm, zero_hbm, out_hbm,
                  srcv, dstv, rows_v, shared):
    c = lax.axis_index("c")
    s = lax.axis_index("s")
    wid = s * 2 + c
    pltpu.sync_copy(src_hbm.at[wid], srcv)
    pltpu.sync_copy(dst_hbm.at[wid], dstv)

    @pl.when(s == 0)
    def _():
        pltpu.sync_copy(zero_hbm, shared)

    plsc.subcore_barrier()
    base = wid * _EW

    def chunk(j, carry):
        pltpu.sync_copy(phi_hbm.at[pl.ds(base + j * CHUNK, CHUNK)], rows_v)
        pltpu.sync_copy(rows_v, shared.at[srcv.at[j]], add=True)
        pltpu.sync_copy(rows_v, shared.at[dstv.at[j]], add=True)
        return carry

    lax.fori_loop(0, NCH, chunk, 0, unroll=False)
    plsc.subcore_barrier()
    pltpu.sync_copy(shared.at[pl.ds(s * _NROW, _NROW)],
                    out_hbm.at[c, pl.ds(s * _NROW, _NROW)])


def _sc_scatter(vals, src3, dst3, zeros):
    import functools
    k = functools.partial(
        pl.kernel,
        out_type=jax.ShapeDtypeStruct((2, NPAD, D), jnp.float32),
        mesh=_sc_mesh(),
        compiler_params=pltpu.CompilerParams(use_tc_tiling_on_sc=False),
        scratch_types=[
            pltpu.VMEM((NCH, CHUNK), jnp.int32),
            pltpu.VMEM((NCH, CHUNK), jnp.int32),
            pltpu.VMEM((CHUNK, D), jnp.float32),
            pltpu.VMEM_SHARED((NPAD, D), jnp.float32),
        ],
    )(_scatter_body)
    return k(vals, src3, dst3, zeros)


def _w32_body(ef_ref, out_ref):
    out_ref[...] = ef_ref[:, 3:4] * jnp.ones((1, D), jnp.float32)


def _w32(ef4):
    return pl.pallas_call(
        _w32_body,
        grid=(EPAD // BE,),
        in_specs=[pl.BlockSpec((BE, 4), lambda i: (i, 0))],
        out_specs=pl.BlockSpec((BE, D), lambda i: (i, 0)),
        out_shape=jax.ShapeDtypeStruct((EPAD, D), jnp.float32),
    )(ef4)


def kernel(bus_type, Line, Yr, Yi, Ysr, Ysi, Yc, P_set, Q_set, V0, Ustart,
           n_nodes_per_graph, params):
    del Ustart, n_nodes_per_graph
    f32 = jnp.float32

    src3 = jnp.asarray(_SRC_PAD.reshape(NW, NCH, CHUNK))
    dst3 = jnp.asarray(_DST_PAD.reshape(NW, NCH, CHUNK))
    zeros = jnp.zeros((NPAD, D), jnp.float32)

    w = Line.reshape(-1).astype(f32)
    ef4 = jnp.stack([Ysr.reshape(-1), Ysi.reshape(-1), Yc.reshape(-1), w], 1)
    ef4 = jnp.pad(ef4, ((0, EPAD - P_TOTAL), (0, 0)))

    padn = lambda x: jnp.pad(x.reshape(-1).astype(f32), (0, NPAD - N))
    slack = padn(bus_type == 1)
    pvm = padn(bus_type == 2)
    valid = jnp.pad(jnp.ones(N, f32), (0, NPAD - N))
    consts = jnp.stack([padn(P_set), padn(Q_set), slack, pvm, valid,
                        jnp.zeros(NPAD, f32), jnp.zeros(NPAD, f32),
                        jnp.zeros(NPAD, f32)], 1)

    vth = jnp.pad(V0[0].astype(f32), ((0, NPAD - N), (0, 0)))
    m = jnp.zeros((NPAD, D), f32)
    ybr, ybi, ybs = _precast(Yr.reshape(N, N), Yi.reshape(N, N))

    deg_parts = _sc_scatter(_w32(ef4), src3, dst3, zeros)
    A = _prep_A(deg_parts)

    losses = []
    for k in range(K):
        iout = _matvec(vth, ybr, ybi, ybs)
        mdst = _sc_gather(m, dst3)
        phi_w = _edge_mlp(mdst, ef4, params['edge'][k])
        agg_parts = _sc_scatter(phi_w, src3, dst3, zeros)
        vth, m, loss = _node_stage(vth, m, iout, consts, A, agg_parts,
                                   params['theta'][k], params['v'][k],
                                   params['m'][k])
        losses.append(loss[0, 0] * (0.96 ** (K - 1 - k)))

    output = vth[None, :N, :]
    phys_loss = jnp.sum(jnp.stack(losses)).reshape(1)
    return (output, phys_loss)


# speedup vs baseline: 1.0183x; 1.0183x over previous
import math
from itertools import combinations

import jax
import jax.numpy as jnp
import numpy as np
from jax import lax
from jax.experimental import pallas as pl
from jax.experimental.pallas import tpu as pltpu
from jax.experimental.pallas import tpu_sc as plsc

D = 32
K = 6
G = 100
_NS = np.arange(G)
N = int(_NS.sum())
P_TOTAL = int(sum(int(n) * (int(n) - 1) // 2 for n in _NS))

NPAD = 4992
BR = 104
NW = 32
CHUNK = 128
NCH = 40
EPAD = NW * NCH * CHUNK
BE = 2048

_pairs = []
_gids = []
for _gi, _n in enumerate(_NS):
    _n = int(_n)
    if _n >= 2:
        _p = np.array(list(combinations(range(_n), 2)), dtype=np.int64)
        _pairs.append(_p)
        _gids.append(np.full(_p.shape[0], _gi, dtype=np.int64))
_pairs = np.concatenate(_pairs, 0)
_gids = np.concatenate(_gids, 0)
_offsets = np.cumsum(_NS) - _NS
_SRC = (_pairs[:, 0] + _offsets[_gids]).astype(np.int32)
_DST = (_pairs[:, 1] + _offsets[_gids]).astype(np.int32)
_SRC_PAD = np.zeros(EPAD, np.int32)
_DST_PAD = np.zeros(EPAD, np.int32)
_SRC_PAD[:P_TOTAL] = _SRC
_DST_PAD[:P_TOTAL] = _DST


def _lrelu(x):
    return jnp.where(x >= 0, x, 0.1 * x)


def _ln(x, g, b):
    mu = jnp.mean(x, axis=-1, keepdims=True)
    var = jnp.mean((x - mu) ** 2, axis=-1, keepdims=True)
    return (x - mu) / jnp.sqrt(var + 1e-5) * g + b


def _mv_body(vth_ref, yr_ref, yi_ref, out_ref):
    v = vth_ref[:, 0:1]
    th = vth_ref[:, 1:2]
    vr = (v * jnp.cos(th))[:N, :]
    vi = (v * jnp.sin(th))[:N, :]
    yr = yr_ref[...]
    yi = yi_ref[...]
    kw = dict(preferred_element_type=jnp.float32)
    t1 = jnp.dot((yr + yi).astype(jnp.bfloat16), vr, **kw)
    t2 = jnp.dot(yr.astype(jnp.bfloat16), vi - vr, **kw)
    t3 = jnp.dot(yi.astype(jnp.bfloat16), vr + vi, **kw)
    out_ref[...] = jnp.concatenate([t1 - t3, t1 + t2], 1)


def _matvec(vth, yr, yi):
    return pl.pallas_call(
        _mv_body,
        grid=(NPAD // BR,),
        in_specs=[
            pl.BlockSpec((NPAD, 2), lambda i: (0, 0)),
            pl.BlockSpec((BR, N), lambda i: (i, 0)),
            pl.BlockSpec((BR, N), lambda i: (i, 0)),
        ],
        out_specs=pl.BlockSpec((BR, 2), lambda i: (i, 0)),
        out_shape=jax.ShapeDtypeStruct((NPAD, 2), jnp.float32),
    )(vth, yr, yi)


def _edge_body(mdst_ref, ef_ref, W1, b1, g1, be1, W2, b2, g2, be2, W3, b3,
               out_ref):
    x = jnp.concatenate([mdst_ref[...], ef_ref[:, 0:3]], 1)
    h = _lrelu(_ln(jnp.dot(x, W1[...], preferred_element_type=jnp.float32)
                   + b1[...], g1[...], be1[...]))
    h = _lrelu(_ln(jnp.dot(h, W2[...], preferred_element_type=jnp.float32)
                   + b2[...], g2[...], be2[...]))
    phi = jnp.dot(h, W3[...], preferred_element_type=jnp.float32) + b3[...]
    out_ref[...] = phi * ef_ref[:, 3:4]


def _edge_mlp(mdst, ef4, p):
    full = lambda s: pl.BlockSpec(s, lambda i: (0,) * len(s))
    return pl.pallas_call(
        _edge_body,
        grid=(EPAD // BE,),
        in_specs=[
            pl.BlockSpec((BE, D), lambda i: (i, 0)),
            pl.BlockSpec((BE, 4), lambda i: (i, 0)),
            full((D + 3, D + 3)), full((1, D + 3)), full((1, D + 3)),
            full((1, D + 3)),
            full((D + 3, D + 3)), full((1, D + 3)), full((1, D + 3)),
            full((1, D + 3)),
            full((D + 3, D)), full((1, D)),
        ],
        out_specs=pl.BlockSpec((BE, D), lambda i: (i, 0)),
        out_shape=jax.ShapeDtypeStruct((EPAD, D), jnp.float32),
    )(mdst, ef4,
      p['W1'], p['b1'].reshape(1, -1), p['g1'].reshape(1, -1),
      p['be1'].reshape(1, -1),
      p['W2'], p['b2'].reshape(1, -1), p['g2'].reshape(1, -1),
      p['be2'].reshape(1, -1),
      p['W3'], p['b3'].reshape(1, -1))


def _prepA_body(degp_ref, a_ref):
    deg = degp_ref[0, :, 0:1] + degp_ref[1, :, 0:1]
    a_ref[...] = 1.0 / jnp.maximum(deg, 1.0)


def _prep_A(deg_parts):
    return pl.pallas_call(
        _prepA_body,
        out_shape=jax.ShapeDtypeStruct((NPAD, 1), jnp.float32),
    )(deg_parts)


def _node_body(vth_ref, m_ref, iout_ref, c_ref, a_ref, aggp_ref,
               tW1, tb1, tg1, tbe1, tW2, tb2, tg2, tbe2, tW3, tb3,
               vW1, vb1, vg1, vbe1, vW2, vb2, vg2, vbe2, vW3, vb3,
               mW1, mb1, mg1, mbe1, mW2, mb2, mg2, mbe2, mW3, mb3,
               vth_out, m_out, loss_out):
    v = vth_ref[:, 0:1]
    th = vth_ref[:, 1:2]
    ir = iout_ref[:, 0:1]
    ii = iout_ref[:, 1:2]
    P = c_ref[:, 0:1]
    Q = c_ref[:, 1:2]
    slack = c_ref[:, 2:3]
    pv = c_ref[:, 3:4]
    valid = c_ref[:, 4:5]
    vr = v * jnp.cos(th)
    vi = v * jnp.sin(th)
    sr = vr * ir + vi * ii
    si = vi * ir - vr * ii
    ok = valid > 0.0
    dP = jnp.where(ok, (1.0 - slack) * (P - sr), 0.0)
    dQ = jnp.where(ok, (1.0 - slack) * (1.0 - pv) * (Q - si), 0.0)
    agg = (aggp_ref[0] + aggp_ref[1]) * a_ref[...]
    feats = jnp.concatenate([v, th, dP, dQ, m_ref[...], agg], 1)

    def blk(W1, b1, g1, be1, W2, b2, g2, be2, W3, b3):
        h = _lrelu(_ln(jnp.dot(feats, W1[...],
                               preferred_element_type=jnp.float32)
                       + b1[...], g1[...], be1[...]))
        h = _lrelu(_ln(jnp.dot(h, W2[...],
                               preferred_element_type=jnp.float32)
                       + b2[...], g2[...], be2[...]))
        return jnp.dot(h, W3[...], preferred_element_type=jnp.float32) + b3[...]

    dth = (1.0 - slack) * blk(tW1, tb1, tg1, tbe1, tW2, tb2, tg2, tbe2,
                              tW3, tb3)
    dv = (1.0 - slack) * (1.0 - pv) * blk(vW1, vb1, vg1, vbe1, vW2, vb2,
                                          vg2, vbe2, vW3, vb3)
    dm = _ln(jnp.tanh(blk(mW1, mb1, mg1, mbe1, mW2, mb2, mg2, mbe2,
                          mW3, mb3)), 1.0, 0.0)
    th_new = jnp.mod(th + dth + math.pi, 2.0 * math.pi) - math.pi
    v_new = jnp.clip(v + dv, 0.4, 1.2)
    vth_out[...] = jnp.concatenate([v_new, th_new], 1)
    m_out[...] = m_ref[...] + dm
    loss_out[...] = jnp.sum((dP * dP + dQ * dQ) * valid,
                            keepdims=True).reshape(1, 1) / float(N)


def _node_stage(vth, m, iout, consts, A, agg_parts, pt, pv_, pm):
    def flat(p):
        return (p['W1'], p['b1'].reshape(1, -1), p['g1'].reshape(1, -1),
                p['be1'].reshape(1, -1),
                p['W2'], p['b2'].reshape(1, -1), p['g2'].reshape(1, -1),
                p['be2'].reshape(1, -1),
                p['W3'], p['b3'].reshape(1, -1))
    return pl.pallas_call(
        _node_body,
        out_shape=(
            jax.ShapeDtypeStruct((NPAD, 2), jnp.float32),
            jax.ShapeDtypeStruct((NPAD, D), jnp.float32),
            jax.ShapeDtypeStruct((1, 1), jnp.float32),
        ),
    )(vth, m, iout, consts, A, agg_parts, *flat(pt), *flat(pv_), *flat(pm))


_EW = NCH * CHUNK
_NROW = NPAD // 16


def _sc_mesh():
    return plsc.VectorSubcoreMesh(core_axis_name="c", subcore_axis_name="s")


def _gather_body(m_hbm, idx_hbm, out_hbm, idx_v, rows_v, sem):
    c = lax.axis_index("c")
    s = lax.axis_index("s")
    wid = s * 2 + c
    pltpu.sync_copy(idx_hbm.at[wid], idx_v)
    base = wid * _EW

    def chunk(j, carry):
        pltpu.async_copy(m_hbm.at[idx_v.at[j]], rows_v, sem).wait()
        pltpu.sync_copy(rows_v, out_hbm.at[pl.ds(base + j * CHUNK, CHUNK)])
        return carry

    lax.fori_loop(0, NCH, chunk, 0, unroll=False)


def _sc_gather(m, dst3):
    import functools
    k = functools.partial(
        pl.kernel,
        out_type=jax.ShapeDtypeStruct((EPAD, D), jnp.float32),
        mesh=_sc_mesh(),
        compiler_params=pltpu.CompilerParams(use_tc_tiling_on_sc=False),
        scratch_types=[
            pltpu.VMEM((NCH, CHUNK), jnp.int32),
            pltpu.VMEM((CHUNK, D), jnp.float32),
            pltpu.SemaphoreType.DMA,
        ],
    )(_gather_body)
    return k(m, dst3)


def _scatter_body(phi_hbm, src_hbm, dst_hbm, zero_hbm, out_hbm,
                  srcv, dstv, rows_v, shared):
    c = lax.axis_index("c")
    s = lax.axis_index("s")
    wid = s * 2 + c
    pltpu.sync_copy(src_hbm.at[wid], srcv)
    pltpu.sync_copy(dst_hbm.at[wid], dstv)

    @pl.when(s == 0)
    def _():
        pltpu.sync_copy(zero_hbm, shared)

    plsc.subcore_barrier()
    base = wid * _EW

    def chunk(j, carry):
        pltpu.sync_copy(phi_hbm.at[pl.ds(base + j * CHUNK, CHUNK)], rows_v)
        pltpu.sync_copy(rows_v, shared.at[srcv.at[j]], add=True)
        pltpu.sync_copy(rows_v, shared.at[dstv.at[j]], add=True)
        return carry

    lax.fori_loop(0, NCH, chunk, 0, unroll=False)
    plsc.subcore_barrier()
    pltpu.sync_copy(shared.at[pl.ds(s * _NROW, _NROW)],
                    out_hbm.at[c, pl.ds(s * _NROW, _NROW)])


def _sc_scatter(vals, src3, dst3, zeros):
    import functools
    k = functools.partial(
        pl.kernel,
        out_type=jax.ShapeDtypeStruct((2, NPAD, D), jnp.float32),
        mesh=_sc_mesh(),
        compiler_params=pltpu.CompilerParams(use_tc_tiling_on_sc=False),
        scratch_types=[
            pltpu.VMEM((NCH, CHUNK), jnp.int32),
            pltpu.VMEM((NCH, CHUNK), jnp.int32),
            pltpu.VMEM((CHUNK, D), jnp.float32),
            pltpu.VMEM_SHARED((NPAD, D), jnp.float32),
        ],
    )(_scatter_body)
    return k(vals, src3, dst3, zeros)


def _w32_body(ef_ref, out_ref):
    out_ref[...] = ef_ref[:, 3:4] * jnp.ones((1, D), jnp.float32)


def _w32(ef4):
    return pl.pallas_call(
        _w32_body,
        grid=(EPAD // BE,),
        in_specs=[pl.BlockSpec((BE, 4), lambda i: (i, 0))],
        out_specs=pl.BlockSpec((BE, D), lambda i: (i, 0)),
        out_shape=jax.ShapeDtypeStruct((EPAD, D), jnp.float32),
    )(ef4)


def kernel(bus_type, Line, Yr, Yi, Ysr, Ysi, Yc, P_set, Q_set, V0, Ustart,
           n_nodes_per_graph, params):
    del Ustart, n_nodes_per_graph
    f32 = jnp.float32

    src3 = jnp.asarray(_SRC_PAD.reshape(NW, NCH, CHUNK))
    dst3 = jnp.asarray(_DST_PAD.reshape(NW, NCH, CHUNK))
    zeros = jnp.zeros((NPAD, D), jnp.float32)

    w = Line.reshape(-1).astype(f32)
    ef4 = jnp.stack([Ysr.reshape(-1), Ysi.reshape(-1), Yc.reshape(-1), w], 1)
    ef4 = jnp.pad(ef4, ((0, EPAD - P_TOTAL), (0, 0)))

    padn = lambda x: jnp.pad(x.reshape(-1).astype(f32), (0, NPAD - N))
    slack = padn(bus_type == 1)
    pvm = padn(bus_type == 2)
    valid = jnp.pad(jnp.ones(N, f32), (0, NPAD - N))
    consts = jnp.stack([padn(P_set), padn(Q_set), slack, pvm, valid,
                        jnp.zeros(NPAD, f32), jnp.zeros(NPAD, f32),
                        jnp.zeros(NPAD, f32)], 1)

    vth = jnp.pad(V0[0].astype(f32), ((0, NPAD - N), (0, 0)))
    m = jnp.zeros((NPAD, D), f32)
    yr = Yr.reshape(N, N)
    yi = Yi.reshape(N, N)

    deg_parts = _sc_scatter(_w32(ef4), src3, dst3, zeros)
    A = _prep_A(deg_parts)

    losses = []
    for k in range(K):
        iout = _matvec(vth, yr, yi)
        mdst = _sc_gather(m, dst3)
        phi_w = _edge_mlp(mdst, ef4, params['edge'][k])
        agg_parts = _sc_scatter(phi_w, src3, dst3, zeros)
        vth, m, loss = _node_stage(vth, m, iout, consts, A, agg_parts,
                                   params['theta'][k], params['v'][k],
                                   params['m'][k])
        losses.append(loss[0, 0] * (0.96 ** (K - 1 - k)))

    output = vth[None, :N, :]
    phys_loss = jnp.sum(jnp.stack(losses)).reshape(1)
    return (output, phys_loss)
